# initial kernel scaffold (unmeasured)
import jax
import jax.numpy as jnp
from jax import lax
from jax.experimental import pallas as pl
from jax.experimental.pallas import tpu as pltpu

N_DEV = 4


def _ag_body(x_ref, out_ref, comm_ref, send_sems, recv_sems, copy_sem):
    my = lax.axis_index("i")
    right = lax.rem(my + 1, N_DEV)
    left = lax.rem(my + N_DEV - 1, N_DEV)

    barrier = pltpu.get_barrier_semaphore()
    for nbr in (left, right):
        pl.semaphore_signal(
            barrier, inc=1, device_id=(nbr,), device_id_type=pl.DeviceIdType.MESH
        )
    pl.semaphore_wait(barrier, 2)

    own = pltpu.make_async_copy(x_ref, out_ref.at[my], copy_sem)
    own.start()
    own.wait()

    for h in range(N_DEV - 1):
        src = x_ref if h == 0 else comm_ref.at[(h - 1) % 2]
        dst_slot = h % 2
        rdma = pltpu.make_async_remote_copy(
            src_ref=src,
            dst_ref=comm_ref.at[dst_slot],
            send_sem=send_sems.at[h],
            recv_sem=recv_sems.at[h],
            device_id=(right,),
            device_id_type=pl.DeviceIdType.MESH,
        )
        rdma.start()
        rdma.wait()

        origin = lax.rem(my + N_DEV - 1 - h, N_DEV)
        cp = pltpu.make_async_copy(comm_ref.at[dst_slot], out_ref.at[origin], copy_sem)
        cp.start()
        cp.wait()


def _all_gather(x16):
    m_per, k = x16.shape
    return pl.pallas_call(
        _ag_body,
        out_shape=jax.ShapeDtypeStruct((N_DEV, m_per, k), x16.dtype),
        in_specs=[pl.BlockSpec(memory_space=pltpu.VMEM)],
        out_specs=pl.BlockSpec(memory_space=pltpu.ANY),
        scratch_shapes=[
            pltpu.VMEM((2, m_per, k), x16.dtype),
            pltpu.SemaphoreType.DMA((N_DEV - 1,)),
            pltpu.SemaphoreType.DMA((N_DEV - 1,)),
            pltpu.SemaphoreType.DMA,
        ],
        compiler_params=pltpu.CompilerParams(collective_id=0),
    )(x16)


_BM = 512


def _mm_body(a_ref, b_ref, o_ref):
    o_ref[...] = jnp.dot(
        a_ref[0], b_ref[...], preferred_element_type=jnp.float32
    ).astype(o_ref.dtype)


def _matmul(a_full, b16):
    n_dev, m_per, k = a_full.shape
    n = b16.shape[1]
    m_total = n_dev * m_per
    per_chunk = m_per // _BM
    return pl.pallas_call(
        _mm_body,
        grid=(m_total // _BM,),
        in_specs=[
            pl.BlockSpec((1, _BM, k), lambda i: (i // per_chunk, i % per_chunk, 0)),
            pl.BlockSpec((k, n), lambda i: (0, 0)),
        ],
        out_specs=pl.BlockSpec((_BM, n), lambda i: (i, 0)),
        out_shape=jax.ShapeDtypeStruct((m_total, n), jnp.bfloat16),
        compiler_params=pltpu.CompilerParams(
            dimension_semantics=("arbitrary",),
        ),
    )(a_full, b16)


def kernel(A, B):
    A16 = A.astype(jnp.bfloat16)
    B16 = B.astype(jnp.bfloat16)
    a_full = _all_gather(A16)
    return _matmul(a_full, B16)


# baseline (device time: 897496 ns/iter reference)
import jax
import jax.numpy as jnp
from jax import lax
from jax.experimental import pallas as pl
from jax.experimental.pallas import tpu as pltpu

N_DEV = 4


def _ag_body(x_ref, out_ref, comm_ref, send_sems, recv_sems, copy_sem):
    my = lax.axis_index("i")
    right = lax.rem(my + 1, N_DEV)
    left = lax.rem(my + N_DEV - 1, N_DEV)

    barrier = pltpu.get_barrier_semaphore()
    for nbr in (left, right):
        pl.semaphore_signal(
            barrier, inc=1, device_id=(nbr,), device_id_type=pl.DeviceIdType.MESH
        )
    pl.semaphore_wait(barrier, 2)

    own = pltpu.make_async_copy(x_ref, out_ref.at[my], copy_sem)
    own.start()
    own.wait()

    for h in range(N_DEV - 1):
        src = x_ref if h == 0 else comm_ref.at[(h - 1) % 2]
        dst_slot = h % 2
        rdma = pltpu.make_async_remote_copy(
            src_ref=src,
            dst_ref=comm_ref.at[dst_slot],
            send_sem=send_sems.at[h],
            recv_sem=recv_sems.at[h],
            device_id=(right,),
            device_id_type=pl.DeviceIdType.MESH,
        )
        rdma.start()
        rdma.wait()

        origin = lax.rem(my + N_DEV - 1 - h, N_DEV)
        cp = pltpu.make_async_copy(comm_ref.at[dst_slot], out_ref.at[origin], copy_sem)
        cp.start()
        cp.wait()


def _all_gather(x16):
    m_per, k = x16.shape
    return pl.pallas_call(
        _ag_body,
        out_shape=jax.ShapeDtypeStruct((N_DEV, m_per, k), x16.dtype),
        in_specs=[pl.BlockSpec(memory_space=pltpu.VMEM)],
        out_specs=pl.BlockSpec(memory_space=pl.ANY),
        scratch_shapes=[
            pltpu.VMEM((2, m_per, k), x16.dtype),
            pltpu.SemaphoreType.DMA((N_DEV - 1,)),
            pltpu.SemaphoreType.DMA((N_DEV - 1,)),
            pltpu.SemaphoreType.DMA,
        ],
        compiler_params=pltpu.CompilerParams(collective_id=0),
    )(x16)


_BM = 512


def _mm_body(a_ref, b_ref, o_ref):
    o_ref[...] = jnp.dot(
        a_ref[0], b_ref[...], preferred_element_type=jnp.float32
    ).astype(o_ref.dtype)


def _matmul(a_full, b16):
    n_dev, m_per, k = a_full.shape
    n = b16.shape[1]
    m_total = n_dev * m_per
    per_chunk = m_per // _BM
    return pl.pallas_call(
        _mm_body,
        grid=(m_total // _BM,),
        in_specs=[
            pl.BlockSpec((1, _BM, k), lambda i: (i // per_chunk, i % per_chunk, 0)),
            pl.BlockSpec((k, n), lambda i: (0, 0)),
        ],
        out_specs=pl.BlockSpec((_BM, n), lambda i: (i, 0)),
        out_shape=jax.ShapeDtypeStruct((m_total, n), jnp.bfloat16),
        compiler_params=pltpu.CompilerParams(
            dimension_semantics=("arbitrary",),
        ),
    )(a_full, b16)


def kernel(A, B):
    A16 = A.astype(jnp.bfloat16)
    B16 = B.astype(jnp.bfloat16)
    a_full = _all_gather(A16)
    return _matmul(a_full, B16)


# device time: 603188 ns/iter; 1.4879x vs baseline; 1.4879x over previous
import jax
import jax.numpy as jnp
from jax import lax
from jax.experimental import pallas as pl
from jax.experimental.pallas import tpu as pltpu

N_DEV = 4


def _ag_body(x_ref, out_ref, send_sems, recv_sems, copy_sem):
    my = lax.axis_index("i")
    right = lax.rem(my + 1, N_DEV)
    left = lax.rem(my + N_DEV - 1, N_DEV)
    half = out_ref.shape[1] // 2

    barrier = pltpu.get_barrier_semaphore()
    for nbr in (left, right):
        pl.semaphore_signal(
            barrier, inc=1, device_id=(nbr,), device_id_type=pl.DeviceIdType.MESH
        )
    pl.semaphore_wait(barrier, 2)

    own = pltpu.make_async_copy(x_ref, out_ref.at[my], copy_sem)
    own.start()
    cw0 = pltpu.make_async_remote_copy(
        src_ref=x_ref,
        dst_ref=out_ref.at[my],
        send_sem=send_sems.at[0],
        recv_sem=recv_sems.at[0],
        device_id=(right,),
        device_id_type=pl.DeviceIdType.MESH,
    )
    ccw0 = pltpu.make_async_remote_copy(
        src_ref=x_ref,
        dst_ref=out_ref.at[my],
        send_sem=send_sems.at[1],
        recv_sem=recv_sems.at[1],
        device_id=(left,),
        device_id_type=pl.DeviceIdType.MESH,
    )
    cw0.start()
    ccw0.start()

    cw0.wait_recv()
    cw1 = pltpu.make_async_remote_copy(
        src_ref=out_ref.at[left, pl.ds(0, half), :],
        dst_ref=out_ref.at[left, pl.ds(0, half), :],
        send_sem=send_sems.at[2],
        recv_sem=recv_sems.at[2],
        device_id=(right,),
        device_id_type=pl.DeviceIdType.MESH,
    )
    cw1.start()

    ccw0.wait_recv()
    ccw1 = pltpu.make_async_remote_copy(
        src_ref=out_ref.at[right, pl.ds(half, half), :],
        dst_ref=out_ref.at[right, pl.ds(half, half), :],
        send_sem=send_sems.at[3],
        recv_sem=recv_sems.at[3],
        device_id=(left,),
        device_id_type=pl.DeviceIdType.MESH,
    )
    ccw1.start()

    cw1.wait_recv()
    ccw1.wait_recv()

    cw0.wait_send()
    ccw0.wait_send()
    cw1.wait_send()
    ccw1.wait_send()
    own.wait()


def _all_gather(x16):
    m_per, k = x16.shape
    return pl.pallas_call(
        _ag_body,
        out_shape=jax.ShapeDtypeStruct((N_DEV, m_per, k), x16.dtype),
        in_specs=[pl.BlockSpec(memory_space=pl.ANY)],
        out_specs=pl.BlockSpec(memory_space=pl.ANY),
        scratch_shapes=[
            pltpu.SemaphoreType.DMA((4,)),
            pltpu.SemaphoreType.DMA((4,)),
            pltpu.SemaphoreType.DMA,
        ],
        compiler_params=pltpu.CompilerParams(collective_id=0),
    )(x16)


_BM = 512


def _mm_body(a_ref, b_ref, o_ref):
    o_ref[...] = jnp.dot(
        a_ref[0], b_ref[...], preferred_element_type=jnp.float32
    ).astype(o_ref.dtype)


def _matmul(a_full, b16):
    n_dev, m_per, k = a_full.shape
    n = b16.shape[1]
    m_total = n_dev * m_per
    per_chunk = m_per // _BM
    return pl.pallas_call(
        _mm_body,
        grid=(m_total // _BM,),
        in_specs=[
            pl.BlockSpec((1, _BM, k), lambda i: (i // per_chunk, i % per_chunk, 0)),
            pl.BlockSpec((k, n), lambda i: (0, 0)),
        ],
        out_specs=pl.BlockSpec((_BM, n), lambda i: (i, 0)),
        out_shape=jax.ShapeDtypeStruct((m_total, n), jnp.bfloat16),
        compiler_params=pltpu.CompilerParams(
            dimension_semantics=("arbitrary",),
        ),
    )(a_full, b16)


def kernel(A, B):
    A16 = A.astype(jnp.bfloat16)
    B16 = B.astype(jnp.bfloat16)
    a_full = _all_gather(A16)
    return _matmul(a_full, B16)


# device time: 473518 ns/iter; 1.8954x vs baseline; 1.2738x over previous
import jax
import jax.numpy as jnp
from jax import lax
from jax.experimental import pallas as pl
from jax.experimental.pallas import tpu as pltpu

N_DEV = 4
_BM = 512


def _fused_body(
    a_ref,
    b_ref,
    out_ref,
    a_full,
    a_tile,
    c_tile,
    a_sems,
    c_sems,
    own_sem,
    send_sems,
    recv_sems,
):
    m_per, k = a_ref.shape
    n = b_ref.shape[1]
    half = m_per // 2
    tpc = m_per // _BM
    htc = tpc // 2
    n_tiles = N_DEV * tpc

    my = lax.axis_index("i")
    right = lax.rem(my + 1, N_DEV)
    left = lax.rem(my + N_DEV - 1, N_DEV)
    diag = lax.rem(my + 2, N_DEV)

    barrier = pltpu.get_barrier_semaphore()
    for nbr in (left, right):
        pl.semaphore_signal(
            barrier, inc=1, device_id=(nbr,), device_id_type=pl.DeviceIdType.MESH
        )
    pl.semaphore_wait(barrier, 2)

    def rdma(src, dst, i, dev):
        return pltpu.make_async_remote_copy(
            src_ref=src,
            dst_ref=dst,
            send_sem=send_sems.at[i],
            recv_sem=recv_sems.at[i],
            device_id=(dev,),
            device_id_type=pl.DeviceIdType.MESH,
        )

    top = (pl.ds(0, half), slice(None))
    bot = (pl.ds(half, half), slice(None))

    cw0h0 = rdma(a_ref.at[top], a_full.at[(my, *top)], 0, right)
    cw0h1 = rdma(a_ref.at[bot], a_full.at[(my, *bot)], 1, right)
    ccw0h0 = rdma(a_ref.at[top], a_full.at[(my, *top)], 2, left)
    ccw0h1 = rdma(a_ref.at[bot], a_full.at[(my, *bot)], 3, left)
    cw1 = rdma(a_full.at[(left, *top)], a_full.at[(left, *top)], 4, right)
    ccw1 = rdma(a_full.at[(right, *bot)], a_full.at[(right, *bot)], 5, left)

    cw0h0.start()
    cw0h1.start()
    ccw0h0.start()
    ccw0h1.start()

    own = pltpu.make_async_copy(a_ref, a_full.at[my], own_sem)
    own.start()
    own.wait()

    def tile_cid(i):
        seg = i // htc
        return jnp.where(
            seg < 2,
            my,
            jnp.where(
                (seg == 2) | (seg == 4),
                left,
                jnp.where((seg == 3) | (seg == 5), right, diag),
            ),
        )

    def tile_trow(i):
        seg = i // htc
        loc = lax.rem(i, htc)
        return jnp.where(
            seg < 2, i, jnp.where((seg == 2) | (seg == 3) | (seg == 6), loc, loc + htc)
        )

    def a_dma(i):
        return pltpu.make_async_copy(
            a_full.at[tile_cid(i), pl.ds(tile_trow(i) * _BM, _BM), :],
            a_tile.at[lax.rem(i, 2)],
            a_sems.at[lax.rem(i, 2)],
        )

    def c_dma(i):
        row0 = tile_cid(i) * m_per + tile_trow(i) * _BM
        return pltpu.make_async_copy(
            c_tile.at[lax.rem(i, 2)],
            out_ref.at[pl.ds(row0, _BM), :],
            c_sems.at[lax.rem(i, 2)],
        )

    def g8():
        cw0h0.wait_recv()
        cw1.start()

    def g12():
        ccw0h0.wait_recv()

    def g16():
        cw0h1.wait_recv()
        ccw0h1.wait_recv()
        ccw1.start()

    guard_ids = (tpc, tpc + htc, 2 * tpc, 3 * tpc, 3 * tpc + htc)
    guard_fns = (g8, g12, g16, cw1.wait_recv, ccw1.wait_recv)

    a_dma(0).start()

    def loop_body(i, _):
        nxt = i + 1
        has_guard = (nxt == guard_ids[0]) | (nxt == guard_ids[1])
        for g in guard_ids[2:]:
            has_guard = has_guard | (nxt == g)

        a_dma(i).wait()
        pl.when((nxt < n_tiles) & jnp.logical_not(has_guard))(
            lambda: a_dma(nxt).start()
        )

        pl.when(i >= 2)(lambda: c_dma(i).wait())
        slot = lax.rem(i, 2)
        c_tile[slot, :, :] = jnp.dot(
            a_tile[slot, :, :], b_ref[:, :], preferred_element_type=jnp.float32
        ).astype(c_tile.dtype)
        c_dma(i).start()

        for gid, gfn in zip(guard_ids, guard_fns):
            pl.when(nxt == gid)(gfn)
        pl.when(has_guard)(lambda: a_dma(nxt).start())
        return ()

    lax.fori_loop(0, n_tiles, loop_body, ())

    c_dma(n_tiles - 2).wait()
    c_dma(n_tiles - 1).wait()
    for r in (cw0h0, cw0h1, ccw0h0, ccw0h1, cw1, ccw1):
        r.wait_send()


def kernel(A, B):
    A16 = A.astype(jnp.bfloat16)
    B16 = B.astype(jnp.bfloat16)
    m_per, k = A16.shape
    n = B16.shape[1]
    c, _ = pl.pallas_call(
        _fused_body,
        out_shape=[
            jax.ShapeDtypeStruct((N_DEV * m_per, n), jnp.bfloat16),
            jax.ShapeDtypeStruct((N_DEV, m_per, k), jnp.bfloat16),
        ],
        in_specs=[
            pl.BlockSpec(memory_space=pl.ANY),
            pl.BlockSpec(memory_space=pltpu.MemorySpace.VMEM),
        ],
        out_specs=[
            pl.BlockSpec(memory_space=pl.ANY),
            pl.BlockSpec(memory_space=pl.ANY),
        ],
        scratch_shapes=[
            pltpu.VMEM((2, _BM, k), jnp.bfloat16),
            pltpu.VMEM((2, _BM, n), jnp.bfloat16),
            pltpu.SemaphoreType.DMA((2,)),
            pltpu.SemaphoreType.DMA((2,)),
            pltpu.SemaphoreType.DMA,
            pltpu.SemaphoreType.DMA((6,)),
            pltpu.SemaphoreType.DMA((6,)),
        ],
        compiler_params=pltpu.CompilerParams(collective_id=0),
    )(A16, B16)
    return c
